# baseline (speedup 1.0000x reference)
import functools

import jax
import jax.numpy as jnp
import numpy as np
from jax import lax
from jax.experimental import pallas as pl
from jax.experimental.pallas import tpu as pltpu
from jax.experimental.pallas import tpu_sc as plsc

_NUM_FIELDS = 26
_FIELD_SIZE = 100000
_EMBED = 16
_ROWS = 16384
_B = _ROWS * _NUM_FIELDS
_NC, _NS = 2, 16
_NW = _NC * _NS
_BPW = _B // _NW
_IDX_ROW = 128
_ROWS_PER_W = _BPW // _IDX_ROW
_CROWS = 13
_NCHUNK = _ROWS_PER_W // _CROWS
_CIDX = _CROWS * _IDX_ROW

_OFFS_TILE = (
    _FIELD_SIZE * (np.arange(_CROWS * _IDX_ROW, dtype=np.int64) % _NUM_FIELDS)
).astype(np.int32).reshape(_CROWS, _IDX_ROW)


def _sc_gather(x2, offs2, table):
    mesh = plsc.VectorSubcoreMesh(core_axis_name="c", subcore_axis_name="s")

    @functools.partial(
        pl.kernel,
        out_type=jax.ShapeDtypeStruct((_B, _EMBED), jnp.float32),
        mesh=mesh,
        compiler_params=pltpu.CompilerParams(use_tc_tiling_on_sc=False),
        scratch_types=[
            pltpu.VMEM((_ROWS_PER_W, _IDX_ROW), jnp.int32),
            pltpu.VMEM((_CROWS, _IDX_ROW), jnp.int32),
            pltpu.VMEM((_CIDX, _EMBED), jnp.float32),
            pltpu.SemaphoreType.DMA,
        ],
    )
    def k(x_hbm, offs_hbm, table_hbm, out_hbm, idx_v, offs_v, rows_v, gsem):
        wid = lax.axis_index("s") * _NC + lax.axis_index("c")
        rbase = wid * _ROWS_PER_W
        obase = wid * _BPW
        pltpu.sync_copy(x_hbm.at[pl.ds(rbase, _ROWS_PER_W)], idx_v)
        pltpu.sync_copy(offs_hbm, offs_v)

        def chunk(c, carry):
            r0 = c * _CROWS
            for t in range(_CROWS):
                for j in range(_IDX_ROW // 16):
                    sl = pl.ds(j * 16, 16)
                    idx_v[r0 + t, sl] = idx_v[r0 + t, sl] + offs_v[t, sl]
            descs = [
                pltpu.async_copy(
                    table_hbm.at[idx_v.at[r0 + t]],
                    rows_v.at[pl.ds(t * _IDX_ROW, _IDX_ROW)],
                    gsem,
                )
                for t in range(_CROWS)
            ]
            for d in descs:
                d.wait()
            pltpu.sync_copy(rows_v, out_hbm.at[pl.ds(obase + c * _CIDX, _CIDX)])
            return carry

        lax.fori_loop(0, _NCHUNK, chunk, 0)

    return k(x2, offs2, table)


def kernel(x, table):
    x2 = x.reshape(_B // _IDX_ROW, _IDX_ROW).astype(jnp.int32)
    offs2 = jnp.asarray(_OFFS_TILE)
    out = _sc_gather(x2, offs2, table)
    return out.reshape(_ROWS, _NUM_FIELDS, _EMBED)
